# baseline (speedup 1.0000x reference)
import jax
import jax.numpy as jnp
from jax import lax
from jax.experimental import pallas as pl
from jax.experimental.pallas import tpu as pltpu
from jax.experimental.pallas import tpu_sc as plsc

B, L, D = 4096, 200, 64
NC, NS = 2, 16
NW = NC * NS
BT = B // NW
NG = 3
NSO = 2
AHEAD = 2


def _emb_body(idx_hbm, tbl_hbm, posb_hbm, out_hbm, idx_v, posl_v, ridx_v,
              rows_v, stage_v, gsem, psem, osem):
    wid = lax.axis_index("s") * NC + lax.axis_index("c")
    b0 = wid * BT
    pltpu.sync_copy(idx_hbm.at[:, pl.ds(b0, BT)], idx_v)

    items = [lax.iota(jnp.int32, 16) + 16 * g for g in range(BT // 16)]

    def fire(l, sg):
        for g in range(BT // 16):
            sl = pl.ds(16 * g, 16)
            ridx_v[sg, sl] = lax.shift_right_logical(idx_v[l, sl], 1)
        pltpu.async_copy(tbl_hbm.at[ridx_v.at[sg]], rows_v.at[sg],
                         gsem.at[sg])
        pltpu.async_copy(posb_hbm.at[l], posl_v.at[sg], psem.at[sg])

    def wait_gather(l, sg):
        pltpu.make_async_copy(tbl_hbm.at[ridx_v.at[sg]], rows_v.at[sg],
                              gsem.at[sg]).wait()
        pltpu.make_async_copy(posb_hbm.at[l], posl_v.at[sg],
                              psem.at[sg]).wait()

    def wait_out(l, so):
        pltpu.make_async_copy(stage_v.at[so], out_hbm.at[l, :, pl.ds(b0, BT)],
                              osem.at[so]).wait()

    for l in range(AHEAD):
        fire(l, l % NG)

    def pos_body(l, carry):
        ln = l + AHEAD

        @pl.when(ln < L)
        def _():
            fire(ln, lax.rem(ln, NG))

        sg = lax.rem(l, NG)
        so = lax.rem(l, NSO)
        wait_gather(l, sg)

        @pl.when(l >= NSO)
        def _():
            wait_out(l - NSO, so)

        par = [(idx_v[l, pl.ds(16 * g, 16)] & 1) * 64
               for g in range(BT // 16)]

        def d_body(d):
            pos_vec = posl_v[sg, pl.ds(d * 16, 16)]
            for g in range(BT // 16):
                col = par[g] + d
                vals = plsc.load_gather(rows_v.at[sg], [items[g], col])
                stage_v[so, d, pl.ds(16 * g, 16)] = vals + pos_vec

        plsc.parallel_loop(0, D, 1, unroll=2)(d_body)
        pltpu.async_copy(stage_v.at[so], out_hbm.at[l, :, pl.ds(b0, BT)],
                         osem.at[so])
        return carry

    lax.fori_loop(0, L, pos_body, 0)

    for k in range(NSO):
        l = L - NSO + k
        wait_out(l, l % NSO)


def kernel(embedding_idx, token_table, pos_table):
    idx_t = embedding_idx.astype(jnp.int32).T
    tbl2 = token_table.reshape(500000, 128)
    posb = jnp.broadcast_to(pos_table[:, :, None],
                            (L, D, 16)).reshape(L, D * 16)
    mesh = plsc.VectorSubcoreMesh(core_axis_name="c", subcore_axis_name="s")
    k = pl.kernel(
        _emb_body,
        out_type=jax.ShapeDtypeStruct((L, D, B), jnp.float32),
        mesh=mesh,
        scratch_types=[
            pltpu.VMEM((L, BT), jnp.int32),
            pltpu.VMEM((NG, D * 16), jnp.float32),
            pltpu.VMEM((NG, BT), jnp.int32),
            pltpu.VMEM((NG, BT, 128), jnp.float32),
            pltpu.VMEM((NSO, D, BT), jnp.float32),
            pltpu.SemaphoreType.DMA((NG,)),
            pltpu.SemaphoreType.DMA((NG,)),
            pltpu.SemaphoreType.DMA((NSO,)),
        ],
        compiler_params=pltpu.CompilerParams(needs_layout_passes=False),
    )
    out_t = k(idx_t, tbl2, posb)
    return out_t.transpose(2, 0, 1)
